# Initial kernel scaffold; baseline (speedup 1.0000x reference)
import jax
import jax.numpy as jnp
from jax.experimental import pallas as pl


def kernel(input_ids, emb_table, hard_norm_scale, proj_w):
    raise NotImplementedError("write your pallas kernel here")


# speedup vs baseline: 2.7245x; 2.7245x over previous
import functools

import jax
import jax.numpy as jnp
from jax import lax
from jax.experimental import pallas as pl
from jax.experimental.pallas import tpu as pltpu
from jax.experimental.pallas import tpu_sc as plsc

_EPS = 1e-06

_NC = 2
_NS = 16
_NW = _NC * _NS


def _precompute_body(emb_ref, scale_ref, w_ref, out_ref):
    x = emb_ref[...]
    var = jnp.mean(x * x, axis=-1, keepdims=True)
    y = x * lax.rsqrt(var + _EPS) * scale_ref[...]
    z = jnp.dot(y, w_ref[...], preferred_element_type=jnp.float32)
    var2 = jnp.mean(z * z, axis=-1, keepdims=True)
    out_ref[...] = z * lax.rsqrt(var2 + _EPS)


def _precompute_table(emb_table, scale2d, proj_w):
    v, _ = emb_table.shape
    f = proj_w.shape[1]
    return pl.pallas_call(
        _precompute_body,
        out_shape=jax.ShapeDtypeStruct((v, f), jnp.float32),
    )(emb_table, scale2d, proj_w)


@functools.lru_cache(maxsize=None)
def _make_gather(b, v, d):
    b_per_w = b // _NW
    chunk = 32
    nchunks = b_per_w // chunk
    mesh = plsc.VectorSubcoreMesh(core_axis_name="c", subcore_axis_name="s")

    @functools.partial(
        pl.kernel,
        mesh=mesh,
        out_type=jax.ShapeDtypeStruct((b, d), jnp.float32),
        scratch_types=[
            pltpu.VMEM((b_per_w,), jnp.int32),
            pltpu.VMEM((chunk, d), jnp.float32),
            pltpu.SemaphoreType.DMA,
        ],
    )
    def gather_kernel(ids_hbm, tab_hbm, out_hbm, idx_v, rows_v, sem):
        wid = lax.axis_index("s") * _NC + lax.axis_index("c")
        base = wid * b_per_w
        pltpu.sync_copy(ids_hbm.at[pl.ds(base, b_per_w)], idx_v)
        for c in range(nchunks):
            off = c * chunk
            pltpu.async_copy(
                tab_hbm.at[idx_v.at[pl.ds(off, chunk)]], rows_v, sem
            ).wait()
            pltpu.sync_copy(rows_v, out_hbm.at[pl.ds(base + off, chunk)])

    return gather_kernel


def kernel(input_ids, emb_table, hard_norm_scale, proj_w):
    bsz, seq = input_ids.shape
    f = proj_w.shape[1]
    ptab = _precompute_table(emb_table, hard_norm_scale.reshape(1, -1), proj_w)
    ids = input_ids.reshape(-1).astype(jnp.int32)
    out = _make_gather(bsz * seq, emb_table.shape[0], f)(ids, ptab)
    return out.reshape(bsz, seq, f)


# speedup vs baseline: 2.8579x; 1.0490x over previous
import functools

import jax
import jax.numpy as jnp
from jax import lax
from jax.experimental import pallas as pl
from jax.experimental.pallas import tpu as pltpu
from jax.experimental.pallas import tpu_sc as plsc

_EPS = 1e-06

_NC = 2
_NS = 16
_NW = _NC * _NS


def _precompute_body(emb_ref, scale_ref, w_ref, out_ref):
    x = emb_ref[...]
    var = jnp.mean(x * x, axis=-1, keepdims=True)
    y = x * lax.rsqrt(var + _EPS) * scale_ref[...]
    z = jnp.dot(y, w_ref[...], preferred_element_type=jnp.float32)
    var2 = jnp.mean(z * z, axis=-1, keepdims=True)
    out_ref[...] = z * lax.rsqrt(var2 + _EPS)


def _precompute_table(emb_table, scale2d, proj_w):
    v, _ = emb_table.shape
    f = proj_w.shape[1]
    return pl.pallas_call(
        _precompute_body,
        out_shape=jax.ShapeDtypeStruct((v, f), jnp.float32),
    )(emb_table, scale2d, proj_w)


@functools.lru_cache(maxsize=None)
def _make_gather(b, v, d):
    b_per_w = b // _NW
    chunk = 16
    nchunks = b_per_w // chunk
    mesh = plsc.VectorSubcoreMesh(core_axis_name="c", subcore_axis_name="s")

    @functools.partial(
        pl.kernel,
        mesh=mesh,
        out_type=jax.ShapeDtypeStruct((b, d), jnp.float32),
        scratch_types=[
            pltpu.VMEM((b_per_w,), jnp.int32),
            pltpu.VMEM((chunk, d), jnp.float32),
            pltpu.VMEM((chunk, d), jnp.float32),
            pltpu.SemaphoreType.DMA,
            pltpu.SemaphoreType.DMA,
        ],
    )
    def gather_kernel(ids_hbm, tab_hbm, out_hbm, idx_v, rows0, rows1, s0, s1):
        wid = lax.axis_index("s") * _NC + lax.axis_index("c")
        base = wid * b_per_w
        pltpu.sync_copy(ids_hbm.at[pl.ds(base, b_per_w)], idx_v)
        bufs = (rows0, rows1)
        sems = (s0, s1)
        copies = [None] * nchunks
        copies[0] = pltpu.async_copy(
            tab_hbm.at[idx_v.at[pl.ds(0, chunk)]], rows0, s0
        )
        for c in range(nchunks):
            if c + 1 < nchunks:
                copies[c + 1] = pltpu.async_copy(
                    tab_hbm.at[idx_v.at[pl.ds((c + 1) * chunk, chunk)]],
                    bufs[(c + 1) % 2],
                    sems[(c + 1) % 2],
                )
            copies[c].wait()
            pltpu.sync_copy(
                bufs[c % 2], out_hbm.at[pl.ds(base + c * chunk, chunk)]
            )

    return gather_kernel


def kernel(input_ids, emb_table, hard_norm_scale, proj_w):
    bsz, seq = input_ids.shape
    f = proj_w.shape[1]
    ptab = _precompute_table(emb_table, hard_norm_scale.reshape(1, -1), proj_w)
    ids = input_ids.reshape(-1).astype(jnp.int32)
    out = _make_gather(bsz * seq, emb_table.shape[0], f)(ids, ptab)
    return out.reshape(bsz, seq, f)
